# initial kernel scaffold (unmeasured)
import jax
import jax.numpy as jnp
from jax import lax
from jax.experimental import pallas as pl
from jax.experimental.pallas import tpu as pltpu

N_DEV = 4


def _gather_body(x_ref, w_ref, xg_ref, wg_ref,
                 copy_sems, sx_sems, rx_sems, sw_sems, rw_sems):
    my = lax.axis_index("i")
    left = (my - 1) % N_DEV
    right = (my + 1) % N_DEV

    barrier = pltpu.get_barrier_semaphore()
    for nbr in (left, right):
        pl.semaphore_signal(barrier, inc=1, device_id=(nbr,),
                            device_id_type=pl.DeviceIdType.MESH)
    pl.semaphore_wait(barrier, 2)

    cx = pltpu.make_async_copy(x_ref, xg_ref.at[my], copy_sems.at[0])
    cw = pltpu.make_async_copy(w_ref, wg_ref.at[my], copy_sems.at[1])
    cx.start()
    cw.start()
    cx.wait()
    cw.wait()

    for h in range(N_DEV - 1):
        src = (my - h) % N_DEV
        rx = pltpu.make_async_remote_copy(
            src_ref=xg_ref.at[src], dst_ref=xg_ref.at[src],
            send_sem=sx_sems.at[h], recv_sem=rx_sems.at[h],
            device_id=(right,), device_id_type=pl.DeviceIdType.MESH)
        rw = pltpu.make_async_remote_copy(
            src_ref=wg_ref.at[src], dst_ref=wg_ref.at[src],
            send_sem=sw_sems.at[h], recv_sem=rw_sems.at[h],
            device_id=(right,), device_id_type=pl.DeviceIdType.MESH)
        rx.start()
        rw.start()
        rx.wait()
        rw.wait()


def _gather(x, w_mat):
    m, kl = x.shape
    _, n = w_mat.shape
    return pl.pallas_call(
        _gather_body,
        out_shape=[
            jax.ShapeDtypeStruct((N_DEV, m, kl), x.dtype),
            jax.ShapeDtypeStruct((N_DEV, kl, n), w_mat.dtype),
        ],
        in_specs=[pl.BlockSpec(memory_space=pltpu.ANY),
                  pl.BlockSpec(memory_space=pltpu.ANY)],
        out_specs=[pl.BlockSpec(memory_space=pltpu.ANY),
                   pl.BlockSpec(memory_space=pltpu.ANY)],
        scratch_shapes=[
            pltpu.SemaphoreType.DMA((2,)),
            pltpu.SemaphoreType.DMA((N_DEV - 1,)),
            pltpu.SemaphoreType.DMA((N_DEV - 1,)),
            pltpu.SemaphoreType.DMA((N_DEV - 1,)),
            pltpu.SemaphoreType.DMA((N_DEV - 1,)),
        ],
        compiler_params=pltpu.CompilerParams(collective_id=0),
    )(x, w_mat)


def _gemm_body(xg_ref, wg_ref, sx_ref, sw_ref, out_ref):
    acc = jnp.dot(xg_ref[0], wg_ref[0], preferred_element_type=jnp.float32)
    for j in range(1, N_DEV):
        acc += jnp.dot(xg_ref[j], wg_ref[j],
                       preferred_element_type=jnp.float32)
    s = sx_ref[0] * sw_ref[0]
    out_ref[...] = jnp.maximum(acc * s, 0.0)


def _gemm(xg, wg, scale_x, scale_w):
    _, m, kl = xg.shape
    _, _, n = wg.shape
    nt = 512
    return pl.pallas_call(
        _gemm_body,
        grid=(n // nt,),
        out_shape=jax.ShapeDtypeStruct((m, n), jnp.float32),
        in_specs=[
            pl.BlockSpec((N_DEV, m, kl), lambda i: (0, 0, 0)),
            pl.BlockSpec((N_DEV, kl, nt), lambda i: (0, 0, i)),
            pl.BlockSpec(memory_space=pltpu.SMEM),
            pl.BlockSpec(memory_space=pltpu.SMEM),
        ],
        out_specs=pl.BlockSpec((m, nt), lambda i: (0, i)),
    )(xg, wg, scale_x, scale_w)


def kernel(x, w_mat, scale_x, scale_w):
    xg, wg = _gather(x, w_mat)
    return _gemm(xg, wg, scale_x, scale_w)


# baseline (device time: 610953 ns/iter reference)
import jax
import jax.numpy as jnp
from jax import lax
from jax.experimental import pallas as pl
from jax.experimental.pallas import tpu as pltpu

N_DEV = 4


def _gather_body(x_ref, w_ref, xg_ref, wg_ref,
                 copy_sems, sx_sems, rx_sems, sw_sems, rw_sems):
    my = lax.axis_index("i")
    left = (my - 1) % N_DEV
    right = (my + 1) % N_DEV

    barrier = pltpu.get_barrier_semaphore()
    for nbr in (left, right):
        pl.semaphore_signal(barrier, inc=1, device_id=(nbr,),
                            device_id_type=pl.DeviceIdType.MESH)
    pl.semaphore_wait(barrier, 2)

    cx = pltpu.make_async_copy(x_ref, xg_ref.at[my], copy_sems.at[0])
    cw = pltpu.make_async_copy(w_ref, wg_ref.at[my], copy_sems.at[1])
    cx.start()
    cw.start()
    cx.wait()
    cw.wait()

    for h in range(N_DEV - 1):
        src = (my - h) % N_DEV
        rx = pltpu.make_async_remote_copy(
            src_ref=xg_ref.at[src], dst_ref=xg_ref.at[src],
            send_sem=sx_sems.at[h], recv_sem=rx_sems.at[h],
            device_id=(right,), device_id_type=pl.DeviceIdType.MESH)
        rw = pltpu.make_async_remote_copy(
            src_ref=wg_ref.at[src], dst_ref=wg_ref.at[src],
            send_sem=sw_sems.at[h], recv_sem=rw_sems.at[h],
            device_id=(right,), device_id_type=pl.DeviceIdType.MESH)
        rx.start()
        rw.start()
        rx.wait()
        rw.wait()


def _gather(x, w_mat):
    m, kl = x.shape
    _, n = w_mat.shape
    return pl.pallas_call(
        _gather_body,
        out_shape=[
            jax.ShapeDtypeStruct((N_DEV, m, kl), x.dtype),
            jax.ShapeDtypeStruct((N_DEV, kl, n), w_mat.dtype),
        ],
        in_specs=[pl.BlockSpec(memory_space=pl.ANY),
                  pl.BlockSpec(memory_space=pl.ANY)],
        out_specs=[pl.BlockSpec(memory_space=pl.ANY),
                   pl.BlockSpec(memory_space=pl.ANY)],
        scratch_shapes=[
            pltpu.SemaphoreType.DMA((2,)),
            pltpu.SemaphoreType.DMA((N_DEV - 1,)),
            pltpu.SemaphoreType.DMA((N_DEV - 1,)),
            pltpu.SemaphoreType.DMA((N_DEV - 1,)),
            pltpu.SemaphoreType.DMA((N_DEV - 1,)),
        ],
        compiler_params=pltpu.CompilerParams(collective_id=0),
    )(x, w_mat)


def _gemm_body(xg_ref, wg_ref, sx_ref, sw_ref, out_ref):
    acc = jnp.dot(xg_ref[0], wg_ref[0], preferred_element_type=jnp.float32)
    for j in range(1, N_DEV):
        acc += jnp.dot(xg_ref[j], wg_ref[j],
                       preferred_element_type=jnp.float32)
    s = sx_ref[0] * sw_ref[0]
    out_ref[...] = jnp.maximum(acc * s, 0.0)


def _gemm(xg, wg, scale_x, scale_w):
    _, m, kl = xg.shape
    _, _, n = wg.shape
    nt = 512
    return pl.pallas_call(
        _gemm_body,
        grid=(n // nt,),
        out_shape=jax.ShapeDtypeStruct((m, n), jnp.float32),
        in_specs=[
            pl.BlockSpec((N_DEV, m, kl), lambda i: (0, 0, 0)),
            pl.BlockSpec((N_DEV, kl, nt), lambda i: (0, 0, i)),
            pl.BlockSpec(memory_space=pltpu.MemorySpace.SMEM),
            pl.BlockSpec(memory_space=pltpu.MemorySpace.SMEM),
        ],
        out_specs=pl.BlockSpec((m, nt), lambda i: (0, i)),
    )(xg, wg, scale_x, scale_w)


def kernel(x, w_mat, scale_x, scale_w):
    x8 = x.astype(jnp.float8_e4m3fn)
    w8 = w_mat.astype(jnp.float8_e4m3fn)
    xg, wg = _gather(x8, w8)
    return _gemm(xg, wg, scale_x, scale_w)


# device time: 401892 ns/iter; 1.5202x vs baseline; 1.5202x over previous
import jax
import jax.numpy as jnp
from jax import lax
from jax.experimental import pallas as pl
from jax.experimental.pallas import tpu as pltpu

N_DEV = 4


def _gather_body(x_ref, w_ref, xg_ref, wg_ref, copy_sems, sems):
    my = lax.axis_index("i")
    left = (my - 1) % N_DEV
    right = (my + 1) % N_DEV
    m, kl = x_ref.shape
    mh = m // 2
    kh = kl // 2

    barrier = pltpu.get_barrier_semaphore()
    for nbr in (left, right):
        pl.semaphore_signal(barrier, inc=1, device_id=(nbr,),
                            device_id_type=pl.DeviceIdType.MESH)
    pl.semaphore_wait(barrier, 2)

    cx = pltpu.make_async_copy(x_ref, xg_ref.at[my], copy_sems.at[0])
    cw = pltpu.make_async_copy(w_ref, wg_ref.at[my], copy_sems.at[1])
    cx.start()
    cw.start()

    def rdma(src, dst, s, r, dev):
        return pltpu.make_async_remote_copy(
            src_ref=src, dst_ref=dst, send_sem=sems.at[s], recv_sem=sems.at[r],
            device_id=(dev,), device_id_type=pl.DeviceIdType.MESH)

    p1 = [
        rdma(x_ref, xg_ref.at[my], 0, 1, right),
        rdma(w_ref, wg_ref.at[my], 2, 3, right),
        rdma(x_ref, xg_ref.at[my], 4, 5, left),
        rdma(w_ref, wg_ref.at[my], 6, 7, left),
    ]
    for d in p1:
        d.start()
    for d in p1:
        d.wait()
    cx.wait()
    cw.wait()

    p2 = [
        rdma(xg_ref.at[right, :mh], xg_ref.at[right, :mh], 8, 9, left),
        rdma(wg_ref.at[right, :kh], wg_ref.at[right, :kh], 10, 11, left),
        rdma(xg_ref.at[left, mh:], xg_ref.at[left, mh:], 12, 13, right),
        rdma(wg_ref.at[left, kh:], wg_ref.at[left, kh:], 14, 15, right),
    ]
    for d in p2:
        d.start()
    for d in p2:
        d.wait()


def _gather(x, w_mat):
    m, kl = x.shape
    _, n = w_mat.shape
    return pl.pallas_call(
        _gather_body,
        out_shape=[
            jax.ShapeDtypeStruct((N_DEV, m, kl), x.dtype),
            jax.ShapeDtypeStruct((N_DEV, kl, n), w_mat.dtype),
        ],
        in_specs=[pl.BlockSpec(memory_space=pl.ANY),
                  pl.BlockSpec(memory_space=pl.ANY)],
        out_specs=[pl.BlockSpec(memory_space=pl.ANY),
                   pl.BlockSpec(memory_space=pl.ANY)],
        scratch_shapes=[
            pltpu.SemaphoreType.DMA((2,)),
            pltpu.SemaphoreType.DMA((16,)),
        ],
        compiler_params=pltpu.CompilerParams(collective_id=0),
    )(x, w_mat)


def _gemm_body(xg_ref, wg_ref, sx_ref, sw_ref, out_ref):
    acc = jnp.dot(xg_ref[0], wg_ref[0], preferred_element_type=jnp.float32)
    for j in range(1, N_DEV):
        acc += jnp.dot(xg_ref[j], wg_ref[j],
                       preferred_element_type=jnp.float32)
    s = sx_ref[0] * sw_ref[0]
    out_ref[...] = jnp.maximum(acc * s, 0.0)


def _gemm(xg, wg, scale_x, scale_w):
    _, m, kl = xg.shape
    _, _, n = wg.shape
    nt = 512
    return pl.pallas_call(
        _gemm_body,
        grid=(n // nt,),
        out_shape=jax.ShapeDtypeStruct((m, n), jnp.float32),
        in_specs=[
            pl.BlockSpec((N_DEV, m, kl), lambda i: (0, 0, 0)),
            pl.BlockSpec((N_DEV, kl, nt), lambda i: (0, 0, i)),
            pl.BlockSpec(memory_space=pltpu.MemorySpace.SMEM),
            pl.BlockSpec(memory_space=pltpu.MemorySpace.SMEM),
        ],
        out_specs=pl.BlockSpec((m, nt), lambda i: (0, i)),
    )(xg, wg, scale_x, scale_w)


def kernel(x, w_mat, scale_x, scale_w):
    x8 = x.astype(jnp.float8_e4m3fn)
    w8 = w_mat.astype(jnp.float8_e4m3fn)
    xg, wg = _gather(x8, w8)
    return _gemm(xg, wg, scale_x, scale_w)
